# baseline (speedup 1.0000x reference)
import functools

import jax
import jax.numpy as jnp
from jax import lax
from jax.experimental import pallas as pl
from jax.experimental.pallas import tpu as pltpu
from jax.experimental.pallas import tpu_sc as plsc

NB_TASKS = 1000
D = 128
BATCH = 16384

NC = 2
NS = 16
L = 16
NW = NC * NS
ROWS_PER_W = BATCH // NW
WORDS_PER_W = ROWS_PER_W * D


def _sc_body(x_hbm, tid_hbm, w_hbm, b_hbm, out_hbm, idx_v, wrows_v, brows_v,
             xbuf_v, sem):
    wid = lax.axis_index("s") * NC + lax.axis_index("c")
    base = wid * WORDS_PER_W

    pltpu.sync_copy(tid_hbm, idx_v)
    pltpu.async_copy(w_hbm.at[idx_v], wrows_v, sem).wait()
    pltpu.async_copy(b_hbm.at[idx_v], brows_v, sem).wait()

    tidv = idx_v[...]
    is0 = tidv == 0
    w_eff = []
    b_eff = []
    for j in range(D // L):
        w_eff.append(jnp.where(is0, 1.0, wrows_v[0, pl.ds(L * j, L)]))
        b_eff.append(jnp.where(is0, 0.0, brows_v[0, pl.ds(L * j, L)]))

    pltpu.sync_copy(x_hbm.at[pl.ds(base, WORDS_PER_W)], xbuf_v)

    def row_body(r, carry):
        off = r * D
        for j in range(D // L):
            sl = pl.ds(off + L * j, L)
            xbuf_v[sl] = xbuf_v[sl] * w_eff[j] + b_eff[j]
        return carry

    lax.fori_loop(0, ROWS_PER_W, row_body, 0)

    pltpu.sync_copy(xbuf_v, out_hbm.at[pl.ds(base, WORDS_PER_W)])


@functools.partial(jax.jit, static_argnames=())
def _sc_affine(x_flat, tid_arr, weight, bias):
    mesh = plsc.VectorSubcoreMesh(core_axis_name="c", subcore_axis_name="s",
                                  num_cores=NC, num_subcores=NS)
    kern = pl.kernel(
        _sc_body,
        out_type=jax.ShapeDtypeStruct((BATCH * D,), jnp.float32),
        mesh=mesh,
        scratch_types=[
            pltpu.VMEM((L,), jnp.int32),
            pltpu.VMEM((L, D), jnp.float32),
            pltpu.VMEM((L, D), jnp.float32),
            pltpu.VMEM((WORDS_PER_W,), jnp.float32),
            pltpu.SemaphoreType.DMA,
        ],
    )
    return kern(x_flat, tid_arr, weight, bias)


def kernel(x, task_id, weight, bias):
    tid_arr = jnp.full((L,), task_id, dtype=jnp.int32)
    out_flat = _sc_affine(x.reshape(-1), tid_arr, weight, bias)
    return out_flat.reshape(BATCH, D)
